# baseline (speedup 1.0000x reference)
import dataclasses

import jax
import jax.numpy as jnp
from jax import lax
from jax.experimental import pallas as pl
from jax.experimental.pallas import tpu as pltpu
from jax.experimental.pallas import tpu_sc as plsc

N = 4194304
NC, NS, L = 2, 16, 16
NW = NC * NS
P = N // NW
B = 512
CH = P // B
STEPS = B // L
ROWS = 16777216 // 16

_mesh = plsc.VectorSubcoreMesh(core_axis_name="c", subcore_axis_name="s")

_cp = pltpu.CompilerParams()
if "needs_layout_passes" in pltpu.CompilerParams.__dataclass_fields__:
    _cp = dataclasses.replace(_cp, needs_layout_passes=False)
if "use_tc_tiling_on_sc" in pltpu.CompilerParams.__dataclass_fields__:
    _cp = dataclasses.replace(_cp, use_tc_tiling_on_sc=False)


class _Buf:
    def __init__(self, xv, yv, zv, wx, wy, wz, pos, idx4, gat, outv,
                 semx, semg):
        self.xv, self.yv, self.zv = xv, yv, zv
        self.wx, self.wy, self.wz = wx, wy, wz
        self.pos, self.idx4, self.gat, self.outv = pos, idx4, gat, outv
        self.semx, self.semg = semx, semg


def _body(xs_hbm, tab_hbm, out_hbm, *scratch):
    a = _Buf(*scratch[0:10], scratch[20], scratch[22])
    b = _Buf(*scratch[10:20], scratch[21], scratch[23])
    wid = lax.axis_index("s") * NC + lax.axis_index("c")
    iota = lax.iota(jnp.int32, L)

    def fire_xs(cg, s):
        base = wid * P + cg * B
        for d, dst in ((0, s.xv), (1, s.yv), (2, s.zv)):
            pltpu.async_copy(xs_hbm.at[pl.ds(d * N + base, B)], dst, s.semx)

    def wait_xs(cg, s):
        base = wid * P + cg * B
        for d, dst in ((0, s.xv), (1, s.yv), (2, s.zv)):
            pltpu.make_async_copy(
                xs_hbm.at[pl.ds(d * N + base, B)], dst, s.semx).wait()

    def idx_pass(s):
        for st in range(STEPS):
            o = st * L
            xf = s.xv[pl.ds(o, L)] * 255.0
            yf = s.yv[pl.ds(o, L)] * 255.0
            zf = s.zv[pl.ds(o, L)] * 255.0
            ix = xf.astype(jnp.int32)
            iy = yf.astype(jnp.int32)
            iz = zf.astype(jnp.int32)
            s.wx[pl.ds(o, L)] = xf - ix.astype(jnp.float32)
            s.wy[pl.ds(o, L)] = yf - iy.astype(jnp.float32)
            s.wz[pl.ds(o, L)] = zf - iz.astype(jnp.float32)
            f000 = (ix << 16) + (iy << 8) + iz
            o16 = f000 & 15
            is15 = o16 == 15
            row = (f000 >> 4) + jnp.where(is15, ROWS, 0)
            s.pos[pl.ds(o, L)] = jnp.where(is15, 7, o16)
            s.idx4[pl.ds(0 * B + o, L)] = row
            s.idx4[pl.ds(1 * B + o, L)] = row + 16
            s.idx4[pl.ds(2 * B + o, L)] = row + 4096
            s.idx4[pl.ds(3 * B + o, L)] = row + 4112

    def fire_g(s):
        pltpu.async_copy(tab_hbm.at[s.idx4], s.gat, s.semg)

    def wait_g(s):
        pltpu.make_async_copy(tab_hbm.at[s.idx4], s.gat, s.semg).wait()

    def combine(s):
        for st in range(STEPS):
            o = st * L
            wx = s.wx[pl.ds(o, L)]
            wy = s.wy[pl.ds(o, L)]
            wz = s.wz[pl.ds(o, L)]
            p0 = s.pos[pl.ds(o, L)]
            p1 = p0 + 1
            r0 = iota + (0 * B + o)
            r1 = iota + (1 * B + o)
            r2 = iota + (2 * B + o)
            r3 = iota + (3 * B + o)
            c000 = plsc.load_gather(s.gat, [r0, p0])
            c001 = plsc.load_gather(s.gat, [r0, p1])
            c010 = plsc.load_gather(s.gat, [r1, p0])
            c011 = plsc.load_gather(s.gat, [r1, p1])
            c100 = plsc.load_gather(s.gat, [r2, p0])
            c101 = plsc.load_gather(s.gat, [r2, p1])
            c110 = plsc.load_gather(s.gat, [r3, p0])
            c111 = plsc.load_gather(s.gat, [r3, p1])
            c00 = c000 + wz * (c001 - c000)
            c01 = c010 + wz * (c011 - c010)
            c10 = c100 + wz * (c101 - c100)
            c11 = c110 + wz * (c111 - c110)
            c0 = c00 + wy * (c01 - c00)
            c1 = c10 + wy * (c11 - c10)
            s.outv[pl.ds(o, L)] = c0 + wx * (c1 - c0)

    def out_chunk(cg, s):
        pltpu.sync_copy(s.outv, out_hbm.at[pl.ds(wid * P + cg * B, B)])

    def half(cg, cur, oth):
        @pl.when(cg + 1 < CH)
        def _():
            fire_xs(cg + 1, oth)

        @pl.when(cg < CH)
        def _():
            wait_xs(cg, cur)
            idx_pass(cur)
            fire_g(cur)

        @pl.when((cg >= 1) & (cg <= CH))
        def _():
            wait_g(oth)
            combine(oth)
            out_chunk(cg - 1, oth)

    fire_xs(0, a)

    @pl.loop(0, CH // 2 + 1)
    def _main(m):
        half(2 * m, a, b)
        half(2 * m + 1, b, a)


def kernel(xs, data):
    xs_t = xs.T.reshape(-1)
    flat = data.reshape(-1)
    shifted = jnp.concatenate([flat[8:], jnp.zeros((8,), jnp.float32)])
    tab = jnp.concatenate(
        [flat.reshape(ROWS, 16), shifted.reshape(ROWS, 16)], axis=0)

    def bufset():
        return [
            pltpu.VMEM((B,), jnp.float32),
            pltpu.VMEM((B,), jnp.float32),
            pltpu.VMEM((B,), jnp.float32),
            pltpu.VMEM((B,), jnp.float32),
            pltpu.VMEM((B,), jnp.float32),
            pltpu.VMEM((B,), jnp.float32),
            pltpu.VMEM((B,), jnp.int32),
            pltpu.VMEM((4 * B,), jnp.int32),
            pltpu.VMEM((4 * B, 16), jnp.float32),
            pltpu.VMEM((B,), jnp.float32),
        ]

    run = pl.kernel(
        _body,
        out_type=jax.ShapeDtypeStruct((N,), jnp.float32),
        mesh=_mesh,
        scratch_types=bufset() + bufset() + [
            pltpu.SemaphoreType.DMA,
            pltpu.SemaphoreType.DMA,
            pltpu.SemaphoreType.DMA,
            pltpu.SemaphoreType.DMA,
        ],
        compiler_params=_cp,
    )
    return run(xs_t, tab)


# speedup vs baseline: 1.7371x; 1.7371x over previous
import dataclasses

import jax
import jax.numpy as jnp
from jax import lax
from jax.experimental import pallas as pl
from jax.experimental.pallas import tpu as pltpu
from jax.experimental.pallas import tpu_sc as plsc

N = 4194304
NC, NS, L = 2, 16, 16
NW = NC * NS
P = N // NW
B = 512
CH = P // B
STEPS = B // L

_mesh = plsc.VectorSubcoreMesh(core_axis_name="c", subcore_axis_name="s")

_cp = pltpu.CompilerParams()
if "needs_layout_passes" in pltpu.CompilerParams.__dataclass_fields__:
    _cp = dataclasses.replace(_cp, needs_layout_passes=False)


class _Buf:
    def __init__(self, xv, yv, zv, wx, wy, wz, idx8, gat, outv, semx, semg):
        self.xv, self.yv, self.zv = xv, yv, zv
        self.wx, self.wy, self.wz = wx, wy, wz
        self.idx8, self.gat, self.outv = idx8, gat, outv
        self.semx, self.semg = semx, semg


def _body(xs_hbm, data_hbm, out_hbm, *scratch):
    a = _Buf(*scratch[0:9], scratch[18], scratch[20])
    b = _Buf(*scratch[9:18], scratch[19], scratch[21])
    wid = lax.axis_index("s") * NC + lax.axis_index("c")

    def fire_xs(cg, s):
        base = wid * P + cg * B
        for d, dst in ((0, s.xv), (1, s.yv), (2, s.zv)):
            pltpu.async_copy(xs_hbm.at[pl.ds(d * N + base, B)], dst, s.semx)

    def wait_xs(cg, s):
        base = wid * P + cg * B
        for d, dst in ((0, s.xv), (1, s.yv), (2, s.zv)):
            pltpu.make_async_copy(
                xs_hbm.at[pl.ds(d * N + base, B)], dst, s.semx).wait()

    def idx_pass(s):
        for st in range(STEPS):
            o = st * L
            xf = s.xv[pl.ds(o, L)] * 255.0
            yf = s.yv[pl.ds(o, L)] * 255.0
            zf = s.zv[pl.ds(o, L)] * 255.0
            ix = xf.astype(jnp.int32)
            iy = yf.astype(jnp.int32)
            iz = zf.astype(jnp.int32)
            s.wx[pl.ds(o, L)] = xf - ix.astype(jnp.float32)
            s.wy[pl.ds(o, L)] = yf - iy.astype(jnp.float32)
            s.wz[pl.ds(o, L)] = zf - iz.astype(jnp.float32)
            f000 = (ix << 16) + (iy << 8) + iz
            s.idx8[pl.ds(0 * B + o, L)] = f000
            s.idx8[pl.ds(1 * B + o, L)] = f000 + 1
            s.idx8[pl.ds(2 * B + o, L)] = f000 + 256
            s.idx8[pl.ds(3 * B + o, L)] = f000 + 257
            s.idx8[pl.ds(4 * B + o, L)] = f000 + 65536
            s.idx8[pl.ds(5 * B + o, L)] = f000 + 65537
            s.idx8[pl.ds(6 * B + o, L)] = f000 + 65792
            s.idx8[pl.ds(7 * B + o, L)] = f000 + 65793

    def fire_g(s):
        pltpu.async_copy(data_hbm.at[s.idx8], s.gat, s.semg)

    def wait_g(s):
        pltpu.make_async_copy(data_hbm.at[s.idx8], s.gat, s.semg).wait()

    def combine(s):
        for st in range(STEPS):
            o = st * L
            wx = s.wx[pl.ds(o, L)]
            wy = s.wy[pl.ds(o, L)]
            wz = s.wz[pl.ds(o, L)]
            c000 = s.gat[pl.ds(0 * B + o, L)]
            c001 = s.gat[pl.ds(1 * B + o, L)]
            c010 = s.gat[pl.ds(2 * B + o, L)]
            c011 = s.gat[pl.ds(3 * B + o, L)]
            c100 = s.gat[pl.ds(4 * B + o, L)]
            c101 = s.gat[pl.ds(5 * B + o, L)]
            c110 = s.gat[pl.ds(6 * B + o, L)]
            c111 = s.gat[pl.ds(7 * B + o, L)]
            c00 = c000 + wz * (c001 - c000)
            c01 = c010 + wz * (c011 - c010)
            c10 = c100 + wz * (c101 - c100)
            c11 = c110 + wz * (c111 - c110)
            c0 = c00 + wy * (c01 - c00)
            c1 = c10 + wy * (c11 - c10)
            s.outv[pl.ds(o, L)] = c0 + wx * (c1 - c0)

    def out_chunk(cg, s):
        pltpu.sync_copy(s.outv, out_hbm.at[pl.ds(wid * P + cg * B, B)])

    def half(cg, cur, oth):
        @pl.when(cg + 1 < CH)
        def _():
            fire_xs(cg + 1, oth)

        @pl.when(cg < CH)
        def _():
            wait_xs(cg, cur)
            idx_pass(cur)
            fire_g(cur)

        @pl.when((cg >= 1) & (cg <= CH))
        def _():
            wait_g(oth)
            combine(oth)
            out_chunk(cg - 1, oth)

    fire_xs(0, a)

    @pl.loop(0, CH // 2 + 1)
    def _main(m):
        half(2 * m, a, b)
        half(2 * m + 1, b, a)


def kernel(xs, data):
    xs_t = xs.T.reshape(-1)
    data_flat = data.reshape(-1)

    def bufset():
        return [
            pltpu.VMEM((B,), jnp.float32),
            pltpu.VMEM((B,), jnp.float32),
            pltpu.VMEM((B,), jnp.float32),
            pltpu.VMEM((B,), jnp.float32),
            pltpu.VMEM((B,), jnp.float32),
            pltpu.VMEM((B,), jnp.float32),
            pltpu.VMEM((8 * B,), jnp.int32),
            pltpu.VMEM((8 * B,), jnp.float32),
            pltpu.VMEM((B,), jnp.float32),
        ]

    run = pl.kernel(
        _body,
        out_type=jax.ShapeDtypeStruct((N,), jnp.float32),
        mesh=_mesh,
        scratch_types=bufset() + bufset() + [
            pltpu.SemaphoreType.DMA,
            pltpu.SemaphoreType.DMA,
            pltpu.SemaphoreType.DMA,
            pltpu.SemaphoreType.DMA,
        ],
        compiler_params=_cp,
    )
    return run(xs_t, data_flat)
